# Initial kernel scaffold; baseline (speedup 1.0000x reference)
import jax
import jax.numpy as jnp
from jax.experimental import pallas as pl


def kernel(x, proto_k, gate):
    raise NotImplementedError("write your pallas kernel here")


# speedup vs baseline: 2.0067x; 2.0067x over previous
import functools
import math

import jax
import jax.numpy as jnp
from jax.experimental import pallas as pl

HIDDEN = 1024
EXPERTS = 64
K = 8
TOKENS = 32768
BLOCK_T = 1024


def _router_block(x_ref, pk_ref, g_ref, w_ref, i_ref):
    logits = jax.lax.dot_general(
        pk_ref[...], x_ref[...],
        dimension_numbers=(((1,), (1,)), ((), ())),
        preferred_element_type=jnp.float32,
        precision=jax.lax.Precision.DEFAULT,
    )
    inv = 1.0 / math.sqrt(HIDDEN)
    vals = jnp.maximum(logits * inv - g_ref[...], 0.0)

    eidx = jax.lax.broadcasted_iota(jnp.int32, (EXPERTS, BLOCK_T), 0)
    w_rows = []
    i_rows = []
    for k in range(K):
        m = jnp.max(vals, axis=0, keepdims=True)
        cand = jnp.where(vals == m, eidx, EXPERTS)
        mi = jnp.min(cand, axis=0, keepdims=True)
        w_rows.append(m)
        i_rows.append(mi)
        if k < K - 1:
            vals = jnp.where(eidx == mi, -1.0, vals)
    wT = jnp.concatenate(w_rows, axis=0)
    iT = jnp.concatenate(i_rows, axis=0)
    w_ref[...] = wT.T
    i_ref[...] = iT.T


@jax.jit
def kernel(x, proto_k, gate):
    gate2d = gate.reshape(EXPERTS, 1)
    grid = (TOKENS // BLOCK_T,)
    w, idx = pl.pallas_call(
        _router_block,
        grid=grid,
        in_specs=[
            pl.BlockSpec((BLOCK_T, HIDDEN), lambda i: (i, 0)),
            pl.BlockSpec((EXPERTS, HIDDEN), lambda i: (0, 0)),
            pl.BlockSpec((EXPERTS, 1), lambda i: (0, 0)),
        ],
        out_specs=[
            pl.BlockSpec((BLOCK_T, K), lambda i: (i, 0)),
            pl.BlockSpec((BLOCK_T, K), lambda i: (i, 0)),
        ],
        out_shape=[
            jax.ShapeDtypeStruct((TOKENS, K), jnp.float32),
            jax.ShapeDtypeStruct((TOKENS, K), jnp.int32),
        ],
    )(x, proto_k, gate2d)
    return (w, idx)


# speedup vs baseline: 2.2028x; 1.0978x over previous
import functools
import math

import jax
import jax.numpy as jnp
from jax.experimental import pallas as pl

HIDDEN = 1024
EXPERTS = 64
K = 8
TOKENS = 32768
BLOCK_T = 2048


def _router_block(x_ref, pk_ref, g_ref, w_ref, i_ref):
    logits = jax.lax.dot_general(
        pk_ref[...], x_ref[...],
        dimension_numbers=(((1,), (1,)), ((), ())),
        preferred_element_type=jnp.float32,
        precision=jax.lax.Precision.DEFAULT,
    )
    inv = 1.0 / math.sqrt(HIDDEN)
    vals = jnp.maximum(logits * inv - g_ref[...], 0.0)

    eidx = jax.lax.broadcasted_iota(jnp.int32, (EXPERTS, BLOCK_T), 0)
    w_rows = []
    i_rows = []
    for k in range(K):
        m = jnp.max(vals, axis=0, keepdims=True)
        cand = jnp.where(vals == m, eidx, EXPERTS)
        mi = jnp.min(cand, axis=0, keepdims=True)
        w_rows.append(m)
        i_rows.append(mi)
        if k < K - 1:
            vals = jnp.where(eidx == mi, -1.0, vals)
    wT = jnp.concatenate(w_rows, axis=0)
    iT = jnp.concatenate(i_rows, axis=0)
    w_ref[...] = wT.T
    i_ref[...] = iT.T


@jax.jit
def kernel(x, proto_k, gate):
    gate2d = gate.reshape(EXPERTS, 1)
    grid = (TOKENS // BLOCK_T,)
    w, idx = pl.pallas_call(
        _router_block,
        grid=grid,
        in_specs=[
            pl.BlockSpec((BLOCK_T, HIDDEN), lambda i: (i, 0)),
            pl.BlockSpec((EXPERTS, HIDDEN), lambda i: (0, 0)),
            pl.BlockSpec((EXPERTS, 1), lambda i: (0, 0)),
        ],
        out_specs=[
            pl.BlockSpec((BLOCK_T, K), lambda i: (i, 0)),
            pl.BlockSpec((BLOCK_T, K), lambda i: (i, 0)),
        ],
        out_shape=[
            jax.ShapeDtypeStruct((TOKENS, K), jnp.float32),
            jax.ShapeDtypeStruct((TOKENS, K), jnp.int32),
        ],
    )(x, proto_k, gate2d)
    return (w, idx)


# speedup vs baseline: 2.2505x; 1.0216x over previous
import functools
import math

import jax
import jax.numpy as jnp
from jax.experimental import pallas as pl

HIDDEN = 1024
EXPERTS = 64
K = 8
TOKENS = 32768
BLOCK_T = 4096


def _router_block(x_ref, pk_ref, g_ref, w_ref, i_ref):
    logits = jax.lax.dot_general(
        pk_ref[...], x_ref[...],
        dimension_numbers=(((1,), (1,)), ((), ())),
        preferred_element_type=jnp.float32,
        precision=jax.lax.Precision.DEFAULT,
    )
    inv = 1.0 / math.sqrt(HIDDEN)
    vals = jnp.maximum(logits * inv - g_ref[...], 0.0)

    eidx = jax.lax.broadcasted_iota(jnp.int32, (EXPERTS, BLOCK_T), 0)
    w_rows = []
    i_rows = []
    for k in range(K):
        m = jnp.max(vals, axis=0, keepdims=True)
        cand = jnp.where(vals == m, eidx, EXPERTS)
        mi = jnp.min(cand, axis=0, keepdims=True)
        w_rows.append(m)
        i_rows.append(mi)
        if k < K - 1:
            vals = jnp.where(eidx == mi, -1.0, vals)
    wT = jnp.concatenate(w_rows, axis=0)
    iT = jnp.concatenate(i_rows, axis=0)
    w_ref[...] = wT.T
    i_ref[...] = iT.T


@jax.jit
def kernel(x, proto_k, gate):
    gate2d = gate.reshape(EXPERTS, 1)
    grid = (TOKENS // BLOCK_T,)
    w, idx = pl.pallas_call(
        _router_block,
        grid=grid,
        in_specs=[
            pl.BlockSpec((BLOCK_T, HIDDEN), lambda i: (i, 0)),
            pl.BlockSpec((EXPERTS, HIDDEN), lambda i: (0, 0)),
            pl.BlockSpec((EXPERTS, 1), lambda i: (0, 0)),
        ],
        out_specs=[
            pl.BlockSpec((BLOCK_T, K), lambda i: (i, 0)),
            pl.BlockSpec((BLOCK_T, K), lambda i: (i, 0)),
        ],
        out_shape=[
            jax.ShapeDtypeStruct((TOKENS, K), jnp.float32),
            jax.ShapeDtypeStruct((TOKENS, K), jnp.int32),
        ],
    )(x, proto_k, gate2d)
    return (w, idx)


# speedup vs baseline: 2.3571x; 1.0474x over previous
import functools
import math

import jax
import jax.numpy as jnp
from jax.experimental import pallas as pl

HIDDEN = 1024
EXPERTS = 64
K = 8
TOKENS = 32768
BLOCK_T = 4096


def _router_block(x_ref, pk_ref, g_ref, w_ref, i_ref):
    logits = jax.lax.dot_general(
        pk_ref[...], x_ref[...],
        dimension_numbers=(((1,), (1,)), ((), ())),
        preferred_element_type=jnp.float32,
        precision=jax.lax.Precision.DEFAULT,
    )
    inv = 1.0 / math.sqrt(HIDDEN)
    vals = jnp.maximum(logits * inv - g_ref[...], 0.0)

    eidxf = jax.lax.broadcasted_iota(
        jnp.int32, (EXPERTS, BLOCK_T), 0).astype(jnp.float32)
    w_rows = []
    i_rows = []
    for k in range(K):
        m = jnp.max(vals, axis=0, keepdims=True)
        cand = jnp.where(vals == m, eidxf, float(EXPERTS))
        mi = jnp.min(cand, axis=0, keepdims=True)
        w_rows.append(m)
        i_rows.append(mi)
        if k < K - 1:
            vals = jnp.where(eidxf == mi, -1.0, vals)
    wT = jnp.concatenate(w_rows, axis=0)
    iT = jnp.concatenate(i_rows, axis=0).astype(jnp.int32)
    w_ref[...] = wT.T
    i_ref[...] = iT.T


@jax.jit
def kernel(x, proto_k, gate):
    gate2d = gate.reshape(EXPERTS, 1)
    grid = (TOKENS // BLOCK_T,)
    w, idx = pl.pallas_call(
        _router_block,
        grid=grid,
        in_specs=[
            pl.BlockSpec((BLOCK_T, HIDDEN), lambda i: (i, 0)),
            pl.BlockSpec((EXPERTS, HIDDEN), lambda i: (0, 0)),
            pl.BlockSpec((EXPERTS, 1), lambda i: (0, 0)),
        ],
        out_specs=[
            pl.BlockSpec((BLOCK_T, K), lambda i: (i, 0)),
            pl.BlockSpec((BLOCK_T, K), lambda i: (i, 0)),
        ],
        out_shape=[
            jax.ShapeDtypeStruct((TOKENS, K), jnp.float32),
            jax.ShapeDtypeStruct((TOKENS, K), jnp.int32),
        ],
    )(x, proto_k, gate2d)
    return (w, idx)


# speedup vs baseline: 2.3613x; 1.0018x over previous
import math

import jax
import jax.numpy as jnp
from jax.experimental import pallas as pl

HIDDEN = 1024
EXPERTS = 64
K = 8
TOKENS = 32768
BLOCK_T = 4096


def _router_block(x_ref, pk_ref, g_ref, w_ref, i_ref):
    logits = jax.lax.dot_general(
        pk_ref[...], x_ref[...],
        dimension_numbers=(((1,), (1,)), ((), ())),
        preferred_element_type=jnp.float32,
        precision=jax.lax.Precision.DEFAULT,
    )
    inv = 1.0 / math.sqrt(HIDDEN)
    vals = jnp.maximum(logits * inv - g_ref[...], 0.0)

    eidxf = jax.lax.broadcasted_iota(
        jnp.int32, (EXPERTS, BLOCK_T), 0).astype(jnp.float32)
    w_rows = []
    i_rows = []
    for k in range(K):
        m = jnp.max(vals, axis=0, keepdims=True)
        cand = jnp.where(vals == m, eidxf, float(EXPERTS))
        mi = jnp.min(cand, axis=0, keepdims=True)
        w_rows.append(m)
        i_rows.append(mi)
        if k < K - 1:
            vals = jnp.where(eidxf == mi, -1.0, vals)
    wT = jnp.concatenate(w_rows, axis=0)
    iT = jnp.concatenate(i_rows, axis=0).astype(jnp.int32)
    w_ref[...] = wT.T
    i_ref[...] = iT.T


@jax.jit
def kernel(x, proto_k, gate):
    gate2d = gate.reshape(EXPERTS, 1)
    grid = (TOKENS // BLOCK_T,)
    w, idx = pl.pallas_call(
        _router_block,
        grid=grid,
        in_specs=[
            pl.BlockSpec((BLOCK_T, HIDDEN), lambda i: (i, 0)),
            pl.BlockSpec((EXPERTS, HIDDEN), lambda i: (0, 0)),
            pl.BlockSpec((EXPERTS, 1), lambda i: (0, 0)),
        ],
        out_specs=[
            pl.BlockSpec((BLOCK_T, K), lambda i: (i, 0)),
            pl.BlockSpec((BLOCK_T, K), lambda i: (i, 0)),
        ],
        out_shape=[
            jax.ShapeDtypeStruct((TOKENS, K), jnp.float32),
            jax.ShapeDtypeStruct((TOKENS, K), jnp.int32),
        ],
    )(x, proto_k, gate2d)
    return (w, idx)
